# baseline (speedup 1.0000x reference)
import functools

import jax
import jax.numpy as jnp
from jax import lax
from jax.experimental import pallas as pl
from jax.experimental.pallas import tpu as pltpu
from jax.experimental.pallas import tpu_sc as plsc

EMBED_D = 64
CHUNK = 512


def _sc_gather(idx_flat, table_flat):
    b_total = idx_flat.shape[0]
    n_table = table_flat.shape[0]
    info = plsc.get_sparse_core_info()
    nc, ns = info.num_cores, info.num_subcores
    nw = nc * ns
    per_w = b_total // nw
    n_chunks = per_w // CHUNK
    n_half = n_chunks // 2
    cwords = CHUNK * EMBED_D

    mesh = plsc.VectorSubcoreMesh(core_axis_name="c", subcore_axis_name="s")

    @functools.partial(
        pl.kernel,
        mesh=mesh,
        out_type=jax.ShapeDtypeStruct((b_total * EMBED_D,), jnp.float32),
        scratch_types=[
            pltpu.VMEM((n_table,), jnp.float32),
            pltpu.VMEM((2, CHUNK), jnp.int32),
            pltpu.VMEM((2, cwords), jnp.float32),
            pltpu.SemaphoreType.DMA,
            pltpu.SemaphoreType.DMA,
        ],
        compiler_params=pltpu.CompilerParams(
            use_tc_tiling_on_sc=False, needs_layout_passes=False
        ),
    )
    def k(table_hbm, idx_hbm, out_hbm, table_v, idx_v, rows_v, sem0, sem1):
        pltpu.sync_copy(table_hbm, table_v)
        wid = lax.axis_index("s") * nc + lax.axis_index("c")
        base0 = wid * per_w
        lane = lax.iota(jnp.int32, 16)
        lane_row = lane * EMBED_D

        def process(g, slot_idx, slot_rows, sem):
            cbase = base0 + g * CHUNK

            @pl.when(g >= 2)
            def _():
                pltpu.make_async_copy(
                    slot_rows, out_hbm.at[pl.ds(0, cwords)], sem
                ).wait()

            pltpu.sync_copy(idx_hbm.at[pl.ds(cbase, CHUNK)], slot_idx)

            def group(t, carry):
                r_vec = slot_idx[pl.ds(t * 16, 16)]
                src_base = r_vec * EMBED_D
                dst_base = t * (16 * EMBED_D) + lane_row
                for j in range(EMBED_D):
                    w = plsc.load_gather(table_v, [src_base + j])
                    plsc.store_scatter(slot_rows, [dst_base + j], w)
                return carry

            lax.fori_loop(0, CHUNK // 16, group, 0)
            pltpu.async_copy(
                slot_rows, out_hbm.at[pl.ds(cbase * EMBED_D, cwords)], sem
            )

        def body(g2, carry):
            process(2 * g2, idx_v.at[0], rows_v.at[0], sem0)
            process(2 * g2 + 1, idx_v.at[1], rows_v.at[1], sem1)
            return carry

        lax.fori_loop(0, n_half, body, 0)

        pltpu.make_async_copy(rows_v.at[0], out_hbm.at[pl.ds(0, cwords)], sem0).wait()
        pltpu.make_async_copy(rows_v.at[1], out_hbm.at[pl.ds(0, cwords)], sem1).wait()

    return k(table_flat, idx_flat)


def kernel(time_idx, table):
    b, s = time_idx.shape
    out = _sc_gather(time_idx.reshape(b * s), table.reshape(-1))
    return out.reshape(b, s, EMBED_D)


# speedup vs baseline: 1.1484x; 1.1484x over previous
import functools

import jax
import jax.numpy as jnp
from jax import lax
from jax.experimental import pallas as pl
from jax.experimental.pallas import tpu as pltpu
from jax.experimental.pallas import tpu_sc as plsc

EMBED_D = 64
CHUNK = 512


def _sc_gather(idx_flat, table_flat):
    b_total = idx_flat.shape[0]
    n_table = table_flat.shape[0]
    info = plsc.get_sparse_core_info()
    nc, ns = info.num_cores, info.num_subcores
    nw = nc * ns
    per_w = b_total // nw
    n_chunks = per_w // CHUNK
    n_half = n_chunks // 2
    cwords = CHUNK * EMBED_D

    mesh = plsc.VectorSubcoreMesh(core_axis_name="c", subcore_axis_name="s")

    @functools.partial(
        pl.kernel,
        mesh=mesh,
        out_type=jax.ShapeDtypeStruct((b_total * EMBED_D,), jnp.float32),
        scratch_types=[
            pltpu.VMEM((n_table,), jnp.float32),
            pltpu.VMEM((2, CHUNK), jnp.int32),
            pltpu.VMEM((2, cwords), jnp.float32),
            pltpu.SemaphoreType.DMA,
            pltpu.SemaphoreType.DMA,
        ],
        compiler_params=pltpu.CompilerParams(
            use_tc_tiling_on_sc=False,
            needs_layout_passes=False,
            disable_bounds_checks=True,
        ),
    )
    def k(table_hbm, idx_hbm, out_hbm, table_v, idx_v, rows_v, sem0, sem1):
        pltpu.sync_copy(table_hbm, table_v)
        wid = lax.axis_index("s") * nc + lax.axis_index("c")
        base0 = wid * per_w
        lane = lax.iota(jnp.int32, 16)
        lane_row = lane * EMBED_D

        def process(g, slot_idx, slot_rows, sem):
            cbase = base0 + g * CHUNK

            @pl.when(g >= 2)
            def _():
                pltpu.make_async_copy(
                    slot_rows, out_hbm.at[pl.ds(0, cwords)], sem
                ).wait()

            pltpu.sync_copy(idx_hbm.at[pl.ds(cbase, CHUNK)], slot_idx)

            @plsc.parallel_loop(0, CHUNK // 16, unroll=2)
            def group(t):
                r_vec = slot_idx[pl.ds(t * 16, 16)]
                src_base = r_vec * EMBED_D
                dst_base = t * (16 * EMBED_D) + lane_row
                for j in range(EMBED_D):
                    w = plsc.load_gather(table_v, [src_base + j])
                    plsc.store_scatter(slot_rows, [dst_base + j], w)
            pltpu.async_copy(
                slot_rows, out_hbm.at[pl.ds(cbase * EMBED_D, cwords)], sem
            )

        def body(g2, carry):
            process(2 * g2, idx_v.at[0], rows_v.at[0], sem0)
            process(2 * g2 + 1, idx_v.at[1], rows_v.at[1], sem1)
            return carry

        lax.fori_loop(0, n_half, body, 0)

        pltpu.make_async_copy(rows_v.at[0], out_hbm.at[pl.ds(0, cwords)], sem0).wait()
        pltpu.make_async_copy(rows_v.at[1], out_hbm.at[pl.ds(0, cwords)], sem1).wait()

    return k(table_flat, idx_flat)


def kernel(time_idx, table):
    b, s = time_idx.shape
    out = _sc_gather(time_idx.reshape(b * s), table.reshape(-1))
    return out.reshape(b, s, EMBED_D)


# speedup vs baseline: 2.9027x; 2.5276x over previous
import functools

import jax
import jax.numpy as jnp
from jax import lax
from jax.experimental import pallas as pl
from jax.experimental.pallas import tpu as pltpu
from jax.experimental.pallas import tpu_sc as plsc

EMBED_D = 64
CHUNK = 512


def _sc_gather(idx_flat, table_flat):
    b_total = idx_flat.shape[0]
    n_table = table_flat.shape[0]
    info = plsc.get_sparse_core_info()
    nc, ns = info.num_cores, info.num_subcores
    nw = nc * ns
    per_w = b_total // nw
    n_chunks = per_w // CHUNK
    n_half = n_chunks // 2
    cwords = CHUNK * EMBED_D

    mesh = plsc.VectorSubcoreMesh(core_axis_name="c", subcore_axis_name="s")

    @functools.partial(
        pl.kernel,
        mesh=mesh,
        out_type=jax.ShapeDtypeStruct((b_total * EMBED_D,), jnp.float32),
        scratch_types=[
            pltpu.VMEM((n_table,), jnp.float32),
            pltpu.VMEM((2, CHUNK), jnp.int32),
            pltpu.VMEM((2, cwords), jnp.float32),
            pltpu.SemaphoreType.DMA,
            pltpu.SemaphoreType.DMA,
        ],
        compiler_params=pltpu.CompilerParams(
            use_tc_tiling_on_sc=False,
            needs_layout_passes=False,
            disable_bounds_checks=True,
        ),
    )
    def k(table_hbm, idx_hbm, out_hbm, table_v, idx_v, rows_v, sem0, sem1):
        pltpu.sync_copy(table_hbm, table_v)
        wid = lax.axis_index("s") * nc + lax.axis_index("c")
        base0 = wid * per_w
        lane = lax.iota(jnp.int32, 16)
        lane_row = lane * EMBED_D

        def process(g, slot_idx, slot_rows, sem):
            cbase = base0 + g * CHUNK

            @pl.when(g >= 2)
            def _():
                pltpu.make_async_copy(
                    slot_rows, out_hbm.at[pl.ds(0, cwords)], sem
                ).wait()

            pltpu.sync_copy(idx_hbm.at[pl.ds(cbase, CHUNK)], slot_idx)

            @plsc.parallel_loop(0, CHUNK // 16, unroll=2)
            def group(t):
                r_vec = slot_idx[pl.ds(t * 16, 16)]
                src_base = r_vec * EMBED_D
                dst_base = t * (16 * EMBED_D) + lane_row
                for j in range(EMBED_D):
                    c = (lane + j) & (EMBED_D - 1)
                    w = plsc.load_gather(table_v, [src_base + c])
                    plsc.store_scatter(slot_rows, [dst_base + c], w)
            pltpu.async_copy(
                slot_rows, out_hbm.at[pl.ds(cbase * EMBED_D, cwords)], sem
            )

        def body(g2, carry):
            process(2 * g2, idx_v.at[0], rows_v.at[0], sem0)
            process(2 * g2 + 1, idx_v.at[1], rows_v.at[1], sem1)
            return carry

        lax.fori_loop(0, n_half, body, 0)

        pltpu.make_async_copy(rows_v.at[0], out_hbm.at[pl.ds(0, cwords)], sem0).wait()
        pltpu.make_async_copy(rows_v.at[1], out_hbm.at[pl.ds(0, cwords)], sem1).wait()

    return k(table_flat, idx_flat)


def kernel(time_idx, table):
    b, s = time_idx.shape
    out = _sc_gather(time_idx.reshape(b * s), table.reshape(-1))
    return out.reshape(b, s, EMBED_D)


# speedup vs baseline: 3.6170x; 1.2461x over previous
import functools

import jax
import jax.numpy as jnp
from jax import lax
from jax.experimental import pallas as pl
from jax.experimental.pallas import tpu as pltpu
from jax.experimental.pallas import tpu_sc as plsc

EMBED_D = 64
CHUNK = 512


def _sc_gather(idx_flat, table_flat):
    b_total = idx_flat.shape[0]
    n_table = table_flat.shape[0]
    info = plsc.get_sparse_core_info()
    nc, ns = info.num_cores, info.num_subcores
    nw = nc * ns
    per_w = b_total // nw
    n_chunks = per_w // CHUNK
    n_half = n_chunks // 2

    mesh = plsc.VectorSubcoreMesh(core_axis_name="c", subcore_axis_name="s")

    @functools.partial(
        pl.kernel,
        mesh=mesh,
        out_type=jax.ShapeDtypeStruct((b_total, EMBED_D), jnp.float32),
        scratch_types=[
            pltpu.VMEM((n_table,), jnp.float32),
            pltpu.VMEM((2, CHUNK), jnp.int32),
            pltpu.VMEM((2, CHUNK, EMBED_D), jnp.float32),
            pltpu.SemaphoreType.DMA,
            pltpu.SemaphoreType.DMA,
        ],
        compiler_params=pltpu.CompilerParams(
            use_tc_tiling_on_sc=False,
            needs_layout_passes=False,
            disable_bounds_checks=True,
        ),
    )
    def k(table_hbm, idx_hbm, out_hbm, table_v, idx_v, rows_v, sem0, sem1):
        pltpu.sync_copy(table_hbm, table_v)
        wid = lax.axis_index("s") * nc + lax.axis_index("c")
        base0 = wid * per_w
        lane = lax.iota(jnp.int32, 16)

        def process(g, slot_idx, slot_rows, sem):
            cbase = base0 + g * CHUNK

            @pl.when(g >= 2)
            def _():
                pltpu.make_async_copy(
                    slot_rows, out_hbm.at[pl.ds(0, CHUNK)], sem
                ).wait()

            pltpu.sync_copy(idx_hbm.at[pl.ds(cbase, CHUNK)], slot_idx)

            @plsc.parallel_loop(0, CHUNK // 16, unroll=2)
            def group(t):
                r_vec = slot_idx[pl.ds(t * 16, 16)]
                src_base = r_vec * EMBED_D
                dst_row = t * 16 + lane
                for j in range(EMBED_D):
                    c = (lane + j) & (EMBED_D - 1)
                    w = plsc.load_gather(table_v, [src_base + c])
                    plsc.store_scatter(slot_rows, [dst_row, c], w)

            pltpu.async_copy(slot_rows, out_hbm.at[pl.ds(cbase, CHUNK)], sem)

        def body(g2, carry):
            process(2 * g2, idx_v.at[0], rows_v.at[0], sem0)
            process(2 * g2 + 1, idx_v.at[1], rows_v.at[1], sem1)
            return carry

        lax.fori_loop(0, n_half, body, 0)

        pltpu.make_async_copy(rows_v.at[0], out_hbm.at[pl.ds(0, CHUNK)], sem0).wait()
        pltpu.make_async_copy(rows_v.at[1], out_hbm.at[pl.ds(0, CHUNK)], sem1).wait()

    return k(table_flat, idx_flat)


def kernel(time_idx, table):
    b, s = time_idx.shape
    out = _sc_gather(time_idx.reshape(b * s), table.reshape(-1))
    return out.reshape(b, s, EMBED_D)


# speedup vs baseline: 5.2670x; 1.4562x over previous
import functools

import jax
import jax.numpy as jnp
from jax import lax
from jax.experimental import pallas as pl
from jax.experimental.pallas import tpu as pltpu
from jax.experimental.pallas import tpu_sc as plsc

EMBED_D = 64
CHUNK = 256


def _sc_gather(idx_flat, table_flat):
    b_total = idx_flat.shape[0]
    n_table = table_flat.shape[0]
    info = plsc.get_sparse_core_info()
    nc, ns = info.num_cores, info.num_subcores
    nw = nc * ns
    per_w = b_total // nw
    n_chunks = per_w // CHUNK
    n_half = n_chunks // 2

    mesh = plsc.VectorSubcoreMesh(core_axis_name="c", subcore_axis_name="s")

    @functools.partial(
        pl.kernel,
        mesh=mesh,
        out_type=jax.ShapeDtypeStruct((b_total, EMBED_D), jnp.float32),
        scratch_types=[
            pltpu.VMEM((n_table,), jnp.float32),
            pltpu.VMEM((CHUNK,), jnp.int32),
            pltpu.VMEM((CHUNK,), jnp.int32),
            pltpu.VMEM((CHUNK, EMBED_D), jnp.float32),
            pltpu.VMEM((CHUNK, EMBED_D), jnp.float32),
            pltpu.SemaphoreType.DMA,
            pltpu.SemaphoreType.DMA,
        ],
        compiler_params=pltpu.CompilerParams(
            use_tc_tiling_on_sc=True,
            needs_layout_passes=False,
            disable_bounds_checks=True,
        ),
    )
    def k(table_hbm, idx_hbm, out_hbm, table_v, idx_v0, idx_v1, rows_v0, rows_v1, sem0, sem1):
        pltpu.sync_copy(table_hbm, table_v)
        wid = lax.axis_index("s") * nc + lax.axis_index("c")
        base0 = wid * per_w
        lane = lax.iota(jnp.int32, 16)

        def process(g, slot_idx, slot_rows, sem):
            cbase = base0 + g * CHUNK

            @pl.when(g >= 2)
            def _():
                pltpu.make_async_copy(
                    slot_rows, out_hbm.at[pl.ds(0, CHUNK)], sem
                ).wait()

            pltpu.sync_copy(idx_hbm.at[pl.ds(cbase, CHUNK)], slot_idx)

            @plsc.parallel_loop(0, CHUNK // 16, unroll=2)
            def group(t):
                r_vec = slot_idx[pl.ds(t * 16, 16)]
                src_base = r_vec * EMBED_D
                dst_row = t * 16 + lane
                for j in range(EMBED_D):
                    c = (lane + j) & (EMBED_D - 1)
                    w = plsc.load_gather(table_v, [src_base + c])
                    plsc.store_scatter(slot_rows, [dst_row, c], w)

            pltpu.async_copy(slot_rows, out_hbm.at[pl.ds(cbase, CHUNK)], sem)

        def body(g2, carry):
            process(2 * g2, idx_v0, rows_v0, sem0)
            process(2 * g2 + 1, idx_v1, rows_v1, sem1)
            return carry

        lax.fori_loop(0, n_half, body, 0)

        pltpu.make_async_copy(rows_v0, out_hbm.at[pl.ds(0, CHUNK)], sem0).wait()
        pltpu.make_async_copy(rows_v1, out_hbm.at[pl.ds(0, CHUNK)], sem1).wait()

    return k(table_flat, idx_flat)


def kernel(time_idx, table):
    b, s = time_idx.shape
    out = _sc_gather(time_idx.reshape(b * s), table.reshape(-1))
    return out.reshape(b, s, EMBED_D)
